# Initial kernel scaffold; baseline (speedup 1.0000x reference)
import functools
import math

import jax
import jax.numpy as jnp
from jax import lax
from jax.experimental import pallas as pl
from jax.experimental.pallas import tpu as pltpu
from jax.experimental.pallas import tpu_sc as plsc

_NODE_TYPES = ('inst', 'data', 'ctrl')
_EDGE_TYPES = (('inst', 'calls', 'data'), ('data', 'flows', 'ctrl'),
               ('ctrl', 'jumps', 'inst'))
_NEG = 0.1

_NC, _NS = 2, 16
_W = _NC * _NS
_N = 10000
_E = 320000
_CF = 64
_EW = _E // _W
_KCH = 80
_NCH = _EW // _KCH
_DENP = 10240


def _sc_agg_kernel(xs_hbm, src_hbm, dst_hbm, asrc_hbm, adst_hbm,
                   acc_out, den_out,
                   asrc_v, adst_v, src_v, dst_v, ex_v, den_v, row_v, zb_v,
                   acc_sh, sem):
    cid = lax.axis_index("c")
    sid = lax.axis_index("s")
    wid = sid * _NC + cid

    pltpu.sync_copy(src_hbm.at[pl.ds(wid * _NCH, _NCH)], src_v)
    pltpu.sync_copy(dst_hbm.at[pl.ds(wid * _NCH, _NCH)], dst_v)
    pltpu.sync_copy(asrc_hbm, asrc_v)
    pltpu.sync_copy(adst_hbm, adst_v)

    zeros16 = jnp.zeros((16,), jnp.float32)

    def _z_den(i, _):
        den_v[pl.ds(i * 16, 16)] = zeros16
        return 0
    lax.fori_loop(0, _DENP // 16, _z_den, 0)

    def _z_zb(i, _):
        zb_v[i, pl.ds(0, 16)] = zeros16
        zb_v[i, pl.ds(16, 16)] = zeros16
        zb_v[i, pl.ds(32, 16)] = zeros16
        zb_v[i, pl.ds(48, 16)] = zeros16
        return 0
    lax.fori_loop(0, 125, _z_zb, 0)

    for j in range(5):
        pltpu.sync_copy(zb_v, acc_sh.at[pl.ds(sid * 625 + j * 125, 125)])
    plsc.subcore_barrier()

    def _logits(i, _):
        for j in range(_KCH // 16):
            sidx = src_v[i, pl.ds(j * 16, 16)]
            didx = dst_v[i, pl.ds(j * 16, 16)]
            a = (plsc.load_gather(asrc_v, [sidx])
                 + plsc.load_gather(adst_v, [didx]))
            a = jnp.where(a > 0, a, _NEG * a)
            ex = jnp.exp(a)
            ex_v[i, pl.ds(j * 16, 16)] = ex
            plsc.addupdate_scatter(den_v, [didx], ex)
        return 0
    lax.fori_loop(0, _NCH, _logits, 0)

    def _rows(i, _):
        pltpu.async_copy(xs_hbm.at[src_v.at[i]], row_v, sem).wait()

        def _scale(e, _c):
            s = ex_v[i, e]
            row_v[e, pl.ds(0, 16)] = row_v[e, pl.ds(0, 16)] * s
            row_v[e, pl.ds(16, 16)] = row_v[e, pl.ds(16, 16)] * s
            row_v[e, pl.ds(32, 16)] = row_v[e, pl.ds(32, 16)] * s
            row_v[e, pl.ds(48, 16)] = row_v[e, pl.ds(48, 16)] * s
            return 0
        lax.fori_loop(0, _KCH, _scale, 0)
        pltpu.sync_copy(row_v, acc_sh.at[dst_v.at[i]], add=True)
        return 0
    lax.fori_loop(0, _NCH, _rows, 0)

    pltpu.sync_copy(den_v, den_out.at[wid])
    plsc.subcore_barrier()
    pltpu.sync_copy(acc_sh.at[pl.ds(sid * 625, 625)],
                    acc_out.at[cid].at[pl.ds(sid * 625, 625)])


@jax.jit
def _sc_agg(xs, src2, dst2, a_src, a_dst):
    mesh = plsc.VectorSubcoreMesh(core_axis_name="c", subcore_axis_name="s")
    f = pl.kernel(
        _sc_agg_kernel,
        out_type=(jax.ShapeDtypeStruct((_NC, _N, _CF), jnp.float32),
                  jax.ShapeDtypeStruct((_W, _DENP), jnp.float32)),
        mesh=mesh,
        scratch_types=[
            pltpu.VMEM((_N,), jnp.float32),
            pltpu.VMEM((_N,), jnp.float32),
            pltpu.VMEM((_NCH, _KCH), jnp.int32),
            pltpu.VMEM((_NCH, _KCH), jnp.int32),
            pltpu.VMEM((_NCH, _KCH), jnp.float32),
            pltpu.VMEM((_DENP,), jnp.float32),
            pltpu.VMEM((_KCH, _CF), jnp.float32),
            pltpu.VMEM((125, _CF), jnp.float32),
            pltpu.VMEM_SHARED((_N, _CF), jnp.float32),
            pltpu.SemaphoreType.DMA,
        ],
    )
    return f(xs, src2, dst2, a_src, a_dst)


def _linear(p, x):
    return x @ p['W'].T + p['b']


def _layer_norm(x, g, b, eps=1e-5):
    mu = x.mean(-1, keepdims=True)
    v = ((x - mu) ** 2).mean(-1, keepdims=True)
    return (x - mu) / jnp.sqrt(v + eps) * g + b


def _han_conv(p, x_dict, ei_dict):
    xp = {nt: _linear(p['proj'][nt], x) for nt, x in x_dict.items()}
    res = {}
    for et, (src2, dst2) in ei_dict.items():
        st, _, dt = et
        s = '__'.join(et)
        a_src = xp[st] @ p['lin_src'][s]
        a_dst = xp[dt] @ p['lin_dst'][s]
        acc, den = _sc_agg(xp[st], src2, dst2, a_src, a_dst)
        den_t = den.sum(0)[:_N]
        msg = (acc[0] + acc[1]) / jnp.maximum(den_t, 1e-30)[:, None]
        res[dt] = jax.nn.relu(msg)
    return res


def _mab(p, x, y, c):
    Wq, Wk, Wv = p['in_W'][:c], p['in_W'][c:2 * c], p['in_W'][2 * c:]
    bq, bk, bv = p['in_b'][:c], p['in_b'][c:2 * c], p['in_b'][2 * c:]
    Q = x @ Wq.T + bq
    K = y @ Wk.T + bk
    V = y @ Wv.T + bv
    A = jax.nn.softmax(jnp.einsum('bqc,bkc->bqk', Q, K) / math.sqrt(c), -1)
    o = _linear(p['out'], jnp.einsum('bqk,bkc->bqc', A, V))
    o = o + x
    o = _layer_norm(o, p['ln1']['g'], p['ln1']['b'])
    o = o + jax.nn.relu(_linear(p['lin'], o))
    return _layer_norm(o, p['ln2']['g'], p['ln2']['b'])


def _set_transformer(p, h, c):
    x = h[None]
    x = _mab(p['enc'], x, x, c)
    z = jax.nn.relu(_linear(p['pma_lin'], x))
    s = jnp.broadcast_to(p['seed'], (x.shape[0], p['seed'].shape[1], c))
    x = _mab(p['pma_mab'], s, z, c)
    x = _mab(p['dec'], x, x, c)
    x = jnp.nan_to_num(x)
    return x.reshape(x.shape[0], -1)


def kernel(x_inst, x_data, x_ctrl, ei_inst_calls_data, ei_data_flows_ctrl,
           ei_ctrl_jumps_inst, params):
    x_dict = {'inst': x_inst, 'data': x_data, 'ctrl': x_ctrl}
    eis = (ei_inst_calls_data, ei_data_flows_ctrl, ei_ctrl_jumps_inst)
    ei_dict = {}
    for et, ei in zip(_EDGE_TYPES, eis):
        src2 = ei[0].astype(jnp.int32).reshape(_E // _KCH, _KCH)
        dst2 = ei[1].astype(jnp.int32).reshape(_E // _KCH, _KCH)
        ei_dict[et] = (src2, dst2)

    h = _han_conv(params['conv1'], x_dict, ei_dict)
    h = {nt: _layer_norm(h[nt], params['norm1'][nt]['g'],
                         params['norm1'][nt]['b']) for nt in h}
    h = _han_conv(params['conv2'], h, ei_dict)
    h = {nt: _layer_norm(h[nt], params['norm2'][nt]['g'],
                         params['norm2'][nt]['b']) for nt in h}
    h = _han_conv(params['conv3'], h, ei_dict)

    hs = []
    for nt in _NODE_TYPES:
        z = _set_transformer(params['st'][nt], h[nt], _CF)
        z = jax.nn.gelu(_linear(params['fc1'][nt], z), approximate=False)
        z = _linear(params['fc2'][nt], z)
        hs.append(z)
    out = jax.nn.relu(_linear(params['out_fc'], jnp.concatenate(hs, -1)))
    return out[0]


# speedup vs baseline: 27.1264x; 27.1264x over previous
import functools
import math

import jax
import jax.numpy as jnp
from jax import lax
from jax.experimental import pallas as pl
from jax.experimental.pallas import tpu as pltpu
from jax.experimental.pallas import tpu_sc as plsc

_NODE_TYPES = ('inst', 'data', 'ctrl')
_EDGE_TYPES = (('inst', 'calls', 'data'), ('data', 'flows', 'ctrl'),
               ('ctrl', 'jumps', 'inst'))
_NEG = 0.1

_NC, _NS = 2, 16
_W = _NC * _NS
_N = 10000
_E = 320000
_CF = 64
_EW = _E // _W
_KCH = 80
_NCH = _EW // _KCH
_NCHP = 128
_EWP = _NCHP * _KCH
_NP = 10240
_DENP = 10240


def _sc_agg_kernel(xs_hbm, src_hbm, dst_hbm, asrc_hbm, adst_hbm,
                   acc_out, den_out,
                   asrc_v, adst_v, src_v, dst_v, ex_v, den_v, row_v, zb_v,
                   acc_sh, sem):
    cid = lax.axis_index("c")
    sid = lax.axis_index("s")
    wid = sid * _NC + cid

    pltpu.sync_copy(src_hbm.at[pl.ds(wid * _NCHP, _NCHP)], src_v)
    pltpu.sync_copy(dst_hbm.at[pl.ds(wid * _NCHP, _NCHP)], dst_v)
    pltpu.sync_copy(asrc_hbm, asrc_v)
    pltpu.sync_copy(adst_hbm, adst_v)

    zeros16 = jnp.zeros((16,), jnp.float32)

    def _z_den(i, _):
        den_v[pl.ds(i * 16, 16)] = zeros16
        return 0
    lax.fori_loop(0, _DENP // 16, _z_den, 0)

    def _z_zb(i, _):
        zb_v[i, pl.ds(0, 16)] = zeros16
        zb_v[i, pl.ds(16, 16)] = zeros16
        zb_v[i, pl.ds(32, 16)] = zeros16
        zb_v[i, pl.ds(48, 16)] = zeros16
        return 0
    lax.fori_loop(0, 128, _z_zb, 0)

    for j in range(5):
        pltpu.sync_copy(zb_v, acc_sh.at[pl.ds(sid * 640 + j * 128, 128)])
    plsc.subcore_barrier()

    zeros16i = jnp.zeros((16,), jnp.int32)

    def _logits(i, _):
        for j in range(_KCH // 16):
            sidx = src_v[i, pl.ds(j * 16, 16)]
            didx = dst_v[i, pl.ds(j * 16, 16)]
            a = (plsc.load_gather(asrc_v, [sidx])
                 + plsc.load_gather(adst_v, [didx]))
            a = jnp.where(a > 0, a, _NEG * a)
            ex = jnp.exp(a)
            ex_v[pl.ds(i * _KCH + j * 16, 16)] = ex
            plsc.addupdate_scatter(den_v, [didx], ex)
        return 0
    lax.fori_loop(0, _NCH, _logits, 0)

    def _rows(i, _):
        pltpu.async_copy(xs_hbm.at[src_v.at[i]], row_v, sem).wait()
        base = i * _KCH

        def _scale(e, _c):
            s = plsc.load_gather(ex_v, [zeros16i + (base + e)])
            row_v[e, pl.ds(0, 16)] = row_v[e, pl.ds(0, 16)] * s
            row_v[e, pl.ds(16, 16)] = row_v[e, pl.ds(16, 16)] * s
            row_v[e, pl.ds(32, 16)] = row_v[e, pl.ds(32, 16)] * s
            row_v[e, pl.ds(48, 16)] = row_v[e, pl.ds(48, 16)] * s
            return 0
        lax.fori_loop(0, _KCH, _scale, 0)
        pltpu.sync_copy(row_v, acc_sh.at[dst_v.at[i]], add=True)
        return 0
    lax.fori_loop(0, _NCH, _rows, 0)

    pltpu.sync_copy(den_v, den_out.at[pl.ds(wid * _DENP, _DENP)])
    plsc.subcore_barrier()
    pltpu.sync_copy(acc_sh.at[pl.ds(sid * 640, 640)],
                    acc_out.at[cid].at[pl.ds(sid * 640, 640)])


@jax.jit
def _sc_agg(xs, src2, dst2, a_src, a_dst):
    mesh = plsc.VectorSubcoreMesh(core_axis_name="c", subcore_axis_name="s")
    f = pl.kernel(
        _sc_agg_kernel,
        out_type=(jax.ShapeDtypeStruct((_NC, _NP, _CF), jnp.float32),
                  jax.ShapeDtypeStruct((_W * _DENP,), jnp.float32)),
        mesh=mesh,
        scratch_types=[
            pltpu.VMEM((_N,), jnp.float32),
            pltpu.VMEM((_N,), jnp.float32),
            pltpu.VMEM((_NCHP, _KCH), jnp.int32),
            pltpu.VMEM((_NCHP, _KCH), jnp.int32),
            pltpu.VMEM((_EW,), jnp.float32),
            pltpu.VMEM((_DENP,), jnp.float32),
            pltpu.VMEM((_KCH, _CF), jnp.float32),
            pltpu.VMEM((128, _CF), jnp.float32),
            pltpu.VMEM_SHARED((_NP, _CF), jnp.float32),
            pltpu.SemaphoreType.DMA,
        ],
        compiler_params=pltpu.CompilerParams(needs_layout_passes=False,
                                             use_tc_tiling_on_sc=False),
    )
    return f(xs, src2, dst2, a_src, a_dst)


def _linear(p, x):
    return x @ p['W'].T + p['b']


def _layer_norm(x, g, b, eps=1e-5):
    mu = x.mean(-1, keepdims=True)
    v = ((x - mu) ** 2).mean(-1, keepdims=True)
    return (x - mu) / jnp.sqrt(v + eps) * g + b


def _han_conv(p, x_dict, ei_dict):
    xp = {nt: _linear(p['proj'][nt], x) for nt, x in x_dict.items()}
    res = {}
    for et, (src2, dst2) in ei_dict.items():
        st, _, dt = et
        s = '__'.join(et)
        a_src = xp[st] @ p['lin_src'][s]
        a_dst = xp[dt] @ p['lin_dst'][s]
        acc, den = _sc_agg(xp[st], src2, dst2, a_src, a_dst)
        den_t = den.reshape(_W, _DENP).sum(0)[:_N]
        msg = (acc[0, :_N] + acc[1, :_N]) / jnp.maximum(den_t, 1e-30)[:, None]
        res[dt] = jax.nn.relu(msg)
    return res


def _mab(p, x, y, c):
    Wq, Wk, Wv = p['in_W'][:c], p['in_W'][c:2 * c], p['in_W'][2 * c:]
    bq, bk, bv = p['in_b'][:c], p['in_b'][c:2 * c], p['in_b'][2 * c:]
    Q = x @ Wq.T + bq
    K = y @ Wk.T + bk
    V = y @ Wv.T + bv
    A = jax.nn.softmax(jnp.einsum('bqc,bkc->bqk', Q, K) / math.sqrt(c), -1)
    o = _linear(p['out'], jnp.einsum('bqk,bkc->bqc', A, V))
    o = o + x
    o = _layer_norm(o, p['ln1']['g'], p['ln1']['b'])
    o = o + jax.nn.relu(_linear(p['lin'], o))
    return _layer_norm(o, p['ln2']['g'], p['ln2']['b'])


def _set_transformer(p, h, c):
    x = h[None]
    x = _mab(p['enc'], x, x, c)
    z = jax.nn.relu(_linear(p['pma_lin'], x))
    s = jnp.broadcast_to(p['seed'], (x.shape[0], p['seed'].shape[1], c))
    x = _mab(p['pma_mab'], s, z, c)
    x = _mab(p['dec'], x, x, c)
    x = jnp.nan_to_num(x)
    return x.reshape(x.shape[0], -1)


def kernel(x_inst, x_data, x_ctrl, ei_inst_calls_data, ei_data_flows_ctrl,
           ei_ctrl_jumps_inst, params):
    x_dict = {'inst': x_inst, 'data': x_data, 'ctrl': x_ctrl}
    eis = (ei_inst_calls_data, ei_data_flows_ctrl, ei_ctrl_jumps_inst)
    ei_dict = {}
    for et, ei in zip(_EDGE_TYPES, eis):
        e32 = ei.astype(jnp.int32).reshape(2, _W, _EW)
        e32 = jnp.pad(e32, ((0, 0), (0, 0), (0, _EWP - _EW)))
        src2 = e32[0].reshape(_W * _NCHP, _KCH)
        dst2 = e32[1].reshape(_W * _NCHP, _KCH)
        ei_dict[et] = (src2, dst2)

    h = _han_conv(params['conv1'], x_dict, ei_dict)
    h = {nt: _layer_norm(h[nt], params['norm1'][nt]['g'],
                         params['norm1'][nt]['b']) for nt in h}
    h = _han_conv(params['conv2'], h, ei_dict)
    h = {nt: _layer_norm(h[nt], params['norm2'][nt]['g'],
                         params['norm2'][nt]['b']) for nt in h}
    h = _han_conv(params['conv3'], h, ei_dict)

    hs = []
    for nt in _NODE_TYPES:
        z = _set_transformer(params['st'][nt], h[nt], _CF)
        z = jax.nn.gelu(_linear(params['fc1'][nt], z), approximate=False)
        z = _linear(params['fc2'][nt], z)
        hs.append(z)
    out = jax.nn.relu(_linear(params['out_fc'], jnp.concatenate(hs, -1)))
    return out[0]
